# baseline (speedup 1.0000x reference)
import numpy as np
import jax
import jax.numpy as jnp
from jax import lax
from jax.experimental import pallas as pl
from jax.experimental.pallas import tpu as pltpu
from jax.experimental.pallas import tpu_sc as plsc

SEARCH = 16
_LANES = 16

_ROTS = ((13, 15, 26, 6), (17, 29, 16, 24))
_M32 = 0xFFFFFFFF


def _tf_np(k0, k1, x0, x1):
    ks = (k0, k1, k0 ^ k1 ^ 0x1BD11BDA)
    x0 = (x0 + ks[0]) & _M32
    x1 = (x1 + ks[1]) & _M32
    for i in range(5):
        for d in _ROTS[i % 2]:
            x0 = (x0 + x1) & _M32
            x1 = ((x1 << d) | (x1 >> (32 - d))) & _M32
            x1 ^= x0
        x0 = (x0 + ks[(i + 1) % 3]) & _M32
        x1 = (x1 + ks[(i + 2) % 3] + i + 1) & _M32
    return x0, x1

_K2A, _K2B = (lambda p: (p[0][1], p[1][1]))(
    tuple(zip(*(_tf_np(0, 42, 0, c) for c in (0, 1)))))
_KS = (_K2A, _K2B, _K2A ^ _K2B ^ 0x1BD11BDA)

_info = plsc.get_sparse_core_info()
_NC, _NS = 1, _info.num_subcores
_NW = _NC * _NS


_CHUNKS = 2


def _onehot_body(out_hbm, buf_v, sem):
    n = out_hbm.shape[1]
    cols = n // _NW
    wid = lax.axis_index("s") * _NC + lax.axis_index("c")
    base = wid * cols

    iota_u = lax.convert_element_type(lax.iota(jnp.int32, _LANES), jnp.uint32)
    base_u = lax.convert_element_type(base, jnp.uint32)

    def group(g, carry):
        c0 = g * _LANES
        lo = base_u + lax.convert_element_type(c0, jnp.uint32) + iota_u
        x0 = jnp.full((_LANES,), np.uint32(_KS[0]), jnp.uint32)
        x1 = lo + np.uint32(_KS[1])
        for i in range(5):
            for d in _ROTS[i % 2]:
                x0 = x0 + x1
                x1 = lax.shift_left(x1, np.uint32(d)) | lax.shift_right_logical(
                    x1, np.uint32(32 - d))
                x1 = x1 ^ x0
            x0 = x0 + np.uint32(_KS[(i + 1) % 3])
            x1 = x1 + np.uint32((_KS[(i + 2) % 3] + i + 1) & _M32)
        idx = (x0 ^ x1) & np.uint32(SEARCH - 1)
        for c in range(SEARCH):
            buf_v[c, pl.ds(c0, _LANES)] = jnp.where(
                idx == np.uint32(c), jnp.float32(1), jnp.float32(0))
        return carry

    ch_cols = cols // _CHUNKS
    ch_groups = ch_cols // _LANES
    copies = []
    for ch in range(_CHUNKS):
        lax.fori_loop(ch * ch_groups, (ch + 1) * ch_groups, group, 0,
                      unroll=False)
        copies.append(pltpu.async_copy(
            buf_v.at[:, pl.ds(ch * ch_cols, ch_cols)],
            out_hbm.at[:, pl.ds(base + ch * ch_cols, ch_cols)], sem))
    for c in copies:
        c.wait()


def kernel(x):
    n = x.shape[0]
    cols = n // _NW
    mesh = plsc.VectorSubcoreMesh(core_axis_name="c", subcore_axis_name="s",
                                  num_cores=1)
    k = pl.kernel(
        _onehot_body,
        out_type=jax.ShapeDtypeStruct((SEARCH, n), x.dtype),
        mesh=mesh,
        scratch_types=[pltpu.VMEM((SEARCH, cols), jnp.float32),
                       pltpu.SemaphoreType.DMA],
        compiler_params=pltpu.CompilerParams(
            needs_layout_passes=False,
            skip_device_barrier=True,
            disable_bounds_checks=True,
            disable_semaphore_checks=True,
        ),
    )
    return k().T


# speedup vs baseline: 1.0128x; 1.0128x over previous
import numpy as np
import jax
import jax.numpy as jnp
from jax import lax
from jax.experimental import pallas as pl
from jax.experimental.pallas import tpu as pltpu
from jax.experimental.pallas import tpu_sc as plsc

SEARCH = 16
_LANES = 16

_ROTS = ((13, 15, 26, 6), (17, 29, 16, 24))
_M32 = 0xFFFFFFFF


def _tf_np(k0, k1, x0, x1):
    ks = (k0, k1, k0 ^ k1 ^ 0x1BD11BDA)
    x0 = (x0 + ks[0]) & _M32
    x1 = (x1 + ks[1]) & _M32
    for i in range(5):
        for d in _ROTS[i % 2]:
            x0 = (x0 + x1) & _M32
            x1 = ((x1 << d) | (x1 >> (32 - d))) & _M32
            x1 ^= x0
        x0 = (x0 + ks[(i + 1) % 3]) & _M32
        x1 = (x1 + ks[(i + 2) % 3] + i + 1) & _M32
    return x0, x1

_K2A, _K2B = (lambda p: (p[0][1], p[1][1]))(
    tuple(zip(*(_tf_np(0, 42, 0, c) for c in (0, 1)))))
_KS = (_K2A, _K2B, _K2A ^ _K2B ^ 0x1BD11BDA)

_info = plsc.get_sparse_core_info()
_NC, _NS = 1, _info.num_subcores
_NW = _NC * _NS


def _onehot_body(out_hbm, buf_v):
    n = out_hbm.shape[1]
    cols = n // _NW
    groups = cols // _LANES
    wid = lax.axis_index("s") * _NC + lax.axis_index("c")
    base = wid * cols

    iota_u = lax.convert_element_type(lax.iota(jnp.int32, _LANES), jnp.uint32)
    base_u = lax.convert_element_type(base, jnp.uint32)

    def group(g, carry):
        c0 = g * _LANES
        lo = base_u + lax.convert_element_type(c0, jnp.uint32) + iota_u
        x0 = jnp.full((_LANES,), np.uint32(_KS[0]), jnp.uint32)
        x1 = lo + np.uint32(_KS[1])
        for i in range(5):
            for d in _ROTS[i % 2]:
                x0 = x0 + x1
                x1 = lax.shift_left(x1, np.uint32(d)) | lax.shift_right_logical(
                    x1, np.uint32(32 - d))
                x1 = x1 ^ x0
            x0 = x0 + np.uint32(_KS[(i + 1) % 3])
            x1 = x1 + np.uint32((_KS[(i + 2) % 3] + i + 1) & _M32)
        idx = (x0 ^ x1) & np.uint32(SEARCH - 1)
        for c in range(SEARCH):
            buf_v[c, pl.ds(c0, _LANES)] = jnp.where(
                idx == np.uint32(c), jnp.float32(1), jnp.float32(0))
        return carry

    lax.fori_loop(0, groups, group, 0, unroll=False)
    pltpu.sync_copy(buf_v, out_hbm.at[:, pl.ds(base, cols)])


def kernel(x):
    n = x.shape[0]
    cols = n // _NW
    mesh = plsc.VectorSubcoreMesh(core_axis_name="c", subcore_axis_name="s",
                                  num_cores=1)
    k = pl.kernel(
        _onehot_body,
        out_type=jax.ShapeDtypeStruct((SEARCH, n), x.dtype),
        mesh=mesh,
        scratch_types=[pltpu.VMEM((SEARCH, cols), jnp.float32)],
        compiler_params=pltpu.CompilerParams(
            needs_layout_passes=False,
            skip_device_barrier=True,
            disable_bounds_checks=True,
            disable_semaphore_checks=True,
        ),
    )
    return k().T
